# baseline (speedup 1.0000x reference)
import functools

import jax
import jax.numpy as jnp
from jax import lax
from jax.experimental import pallas as pl
from jax.experimental.pallas import tpu as pltpu
from jax.experimental.pallas import tpu_sc as plsc


def _tc_body(x_ref, w_ref, b_ref, dis_ref, aw_ref, ab_ref, e_ref,
             t_ref, supp_ref):
    xb = x_ref[...]
    w = w_ref[0]
    t = lax.dot_general(xb, w, (((0,), (0,)), ((), ())),
                        preferred_element_type=jnp.float32)
    t_ref[...] = t + b_ref[0]
    d = dis_ref[...]
    sc = jnp.tanh(jnp.dot(d, aw_ref[...], preferred_element_type=jnp.float32)
                  + ab_ref[...]) + 1.0
    sup = 2.0 * jax.nn.sigmoid(-d * sc)
    supp_ref[...] = jnp.dot(sup, e_ref[...],
                            preferred_element_type=jnp.float32)


def _tc_tables(xp, wst, bst, disp, aw, ab, ee, *, C, O, K, NPAD, BN):
    nb = NPAD // BN
    return pl.pallas_call(
        _tc_body,
        grid=(2, nb),
        in_specs=[
            pl.BlockSpec((C, BN), lambda s, i: (0, i)),
            pl.BlockSpec((1, C, O), lambda s, i: (s, 0, 0)),
            pl.BlockSpec((1, 1, O), lambda s, i: (s, 0, 0)),
            pl.BlockSpec((BN, K), lambda s, i: (i, 0)),
            pl.BlockSpec((K, 1), lambda s, i: (0, 0)),
            pl.BlockSpec((1, 1), lambda s, i: (0, 0)),
            pl.BlockSpec((K, 16 * K), lambda s, i: (0, 0)),
        ],
        out_specs=[
            pl.BlockSpec((BN, O), lambda s, i: (s * nb + i, 0)),
            pl.BlockSpec((BN, 16 * K), lambda s, i: (i, 0)),
        ],
        out_shape=[
            jax.ShapeDtypeStruct((2 * NPAD, O), jnp.float32),
            jax.ShapeDtypeStruct((NPAD, 16 * K), jnp.float32),
        ],
    )(xp, wst, bst, disp, aw, ab, ee)


def _make_sc_combine(*, O, K, NPAD, CH, NBUF, NPT0, NPT1):
    K2 = 2 * K
    GC = CH * K2
    NCH0 = NPT0 // CH
    NCH1 = NPT1 // CH
    OC = O // 16

    nc, ns = 2, 16
    mesh = plsc.VectorSubcoreMesh(core_axis_name="c", subcore_axis_name="s",
                                  num_cores=nc, num_subcores=ns)

    def body(tbl_hbm, idx_hbm, supp_hbm, out_hbm, idx_v, *rest):
        rows = rest[0:NBUF]
        sbs = rest[NBUF:2 * NBUF]
        obs = rest[2 * NBUF:3 * NBUF]
        sems = rest[3 * NBUF:4 * NBUF]
        semos = rest[4 * NBUF:5 * NBUF]

        cid = lax.axis_index("c")
        sid = lax.axis_index("s")
        base = lax.select(cid == 0, sid * NPT0, ns * NPT0 + sid * NPT1)
        nch = lax.select(cid == 0, NCH0, NCH1)
        trips = lax.select(cid == 0, NCH0 // NBUF, NCH1 // NBUF)

        @pl.when(cid == 0)
        def _():
            pltpu.sync_copy(idx_hbm.at[pl.ds(base * K2, NPT0 * K2)],
                            idx_v.at[pl.ds(0, NPT0 * K2)])

        @pl.when(cid != 0)
        def _():
            pltpu.sync_copy(idx_hbm.at[pl.ds(base * K2, NPT1 * K2)],
                            idx_v.at[pl.ds(0, NPT1 * K2)])

        NS_G = GC // 128

        def fetch(c, b, do):
            ds = [pltpu.make_async_copy(
                      tbl_hbm.at[idx_v.at[pl.ds(c * GC + i * 128, 128)]],
                      rows[b].at[pl.ds(i * 128, 128)], sems[b])
                  for i in range(NS_G)]
            ds.append(pltpu.make_async_copy(
                supp_hbm.at[pl.ds(base + c * CH, CH)], sbs[b], sems[b]))
            for d in ds:
                d.start() if do == "start" else d.wait()

        for b in range(NBUF):
            fetch(b, b, "start")

        def outcopy(c, b):
            return pltpu.make_async_copy(
                obs[b], out_hbm.at[pl.ds(base + c * CH, CH)], semos[b])

        def compute_chunk(rv, sv, ov):
            for j in range(CH):
                rbase = j * K2

                def kbody(k, accs):
                    s = sv[j, pl.ds(k * 16, 16)]
                    out = []
                    for oc in range(OC):
                        u = rv[rbase + k, pl.ds(oc * 16, 16)]
                        v = rv[rbase + K + k, pl.ds(oc * 16, 16)]
                        f = jnp.maximum(u + v, 0.0) * s
                        out.append(jnp.maximum(accs[oc], f))
                    return tuple(out)

                accs = lax.fori_loop(
                    0, K, kbody,
                    tuple(jnp.zeros((16,), jnp.float32) for _ in range(OC)))
                for oc in range(OC):
                    ov[j, pl.ds(oc * 16, 16)] = accs[oc]

        def outer(it, carry):
            for b in range(NBUF):
                c = it * NBUF + b
                fetch(c, b, "wait")

                @pl.when(c >= NBUF)
                def _():
                    outcopy(c - NBUF, b).wait()

                compute_chunk(rows[b], sbs[b], obs[b])
                outcopy(c, b).start()

                @pl.when(c + NBUF < nch)
                def _():
                    fetch(c + NBUF, b, "start")

            return carry

        lax.fori_loop(0, trips, outer, 0)
        for b in range(NBUF):
            outcopy(nch - NBUF + b, b).wait()

    return functools.partial(
        pl.kernel,
        out_type=jax.ShapeDtypeStruct((NPAD, O), jnp.float32),
        mesh=mesh,
        scratch_types=(
            [pltpu.VMEM((NPT0 * K2,), jnp.int32)]
            + [pltpu.VMEM((GC, O), jnp.float32) for _ in range(NBUF)]
            + [pltpu.VMEM((CH, 16 * K), jnp.float32) for _ in range(NBUF)]
            + [pltpu.VMEM((CH, O), jnp.float32) for _ in range(NBUF)]
            + [pltpu.SemaphoreType.DMA for _ in range(2 * NBUF)]
        ),
    )(body)


def kernel(x, edge_index, pos, dis, W, b, att_W, att_b):
    del pos
    B, C, N, _ = x.shape
    K = edge_index.shape[-1]
    O = W.shape[0]
    NS = 16
    CH = 4
    NBUF = 4
    QUANT = 2 * NS * CH * NBUF
    NPAD = ((N + QUANT - 1) // QUANT) * QUANT
    BN = NPAD // 4
    NT = NPAD // NS
    NPT0 = ((NT * 4) // (5 * CH * NBUF)) * (CH * NBUF)
    NPT1 = NT - NPT0

    X = x[0, :, :, 0]
    xp = jnp.pad(X, ((0, 0), (0, NPAD - N)))
    disp = jnp.pad(dis[0], ((0, NPAD - N), (0, 0)))
    i1 = edge_index[1, 0].astype(jnp.int32)
    i0 = edge_index[0, 0].astype(jnp.int32)
    idx = jnp.concatenate([i1, i0 + NPAD], axis=1)
    idxp = jnp.pad(idx, ((0, NPAD - N), (0, 0))).reshape(-1)

    W1, W2 = W[:, :C], W[:, C:]
    wst = jnp.stack([(W1 - W2).T, W2.T])
    bst = jnp.stack([b[None, :], jnp.zeros((1, O), jnp.float32)])
    aw = att_W[0].reshape(K, 1).astype(jnp.float32)
    ab = att_b.reshape(1, 1).astype(jnp.float32)
    ee = jnp.repeat(jnp.eye(K, dtype=jnp.float32), 16, axis=1)

    tbl, supp = _tc_tables(xp, wst, bst, disp, aw, ab, ee,
                           C=C, O=O, K=K, NPAD=NPAD, BN=BN)
    out = _make_sc_combine(O=O, K=K, NPAD=NPAD, CH=CH, NBUF=NBUF,
                           NPT0=NPT0, NPT1=NPT1)(
        tbl, idxp, supp)
    return out[:N].T[None, :, :, None]


# speedup vs baseline: 1.0628x; 1.0628x over previous
import functools

import jax
import jax.numpy as jnp
from jax import lax
from jax.experimental import pallas as pl
from jax.experimental.pallas import tpu as pltpu
from jax.experimental.pallas import tpu_sc as plsc


def _tc_body(x_ref, w_ref, b_ref, dis_ref, aw_ref, ab_ref, e_ref,
             t_ref, supp_ref):
    xb = x_ref[...]
    w = w_ref[0]
    t = lax.dot_general(xb, w, (((0,), (0,)), ((), ())),
                        preferred_element_type=jnp.float32)
    t_ref[...] = t + b_ref[0]
    d = dis_ref[...]
    sc = jnp.tanh(jnp.dot(d, aw_ref[...], preferred_element_type=jnp.float32)
                  + ab_ref[...]) + 1.0
    sup = 2.0 * jax.nn.sigmoid(-d * sc)
    supp_ref[...] = jnp.dot(sup, e_ref[...],
                            preferred_element_type=jnp.float32)


def _tc_tables(xp, wst, bst, disp, aw, ab, ee, *, C, O, K, NPAD, BN):
    nb = NPAD // BN
    return pl.pallas_call(
        _tc_body,
        grid=(2, nb),
        in_specs=[
            pl.BlockSpec((C, BN), lambda s, i: (0, i)),
            pl.BlockSpec((1, C, O), lambda s, i: (s, 0, 0)),
            pl.BlockSpec((1, 1, O), lambda s, i: (s, 0, 0)),
            pl.BlockSpec((BN, K), lambda s, i: (i, 0)),
            pl.BlockSpec((K, 1), lambda s, i: (0, 0)),
            pl.BlockSpec((1, 1), lambda s, i: (0, 0)),
            pl.BlockSpec((K, 16 * K), lambda s, i: (0, 0)),
        ],
        out_specs=[
            pl.BlockSpec((BN, O), lambda s, i: (s * nb + i, 0)),
            pl.BlockSpec((BN, 16 * K), lambda s, i: (i, 0)),
        ],
        out_shape=[
            jax.ShapeDtypeStruct((2 * NPAD, O), jnp.float32),
            jax.ShapeDtypeStruct((NPAD, 16 * K), jnp.float32),
        ],
    )(xp, wst, bst, disp, aw, ab, ee)


def _make_sc_combine(*, O, K, NPAD, CH, NBUF, NPT0, NPT1):
    K2 = 2 * K
    GC = CH * K2
    NCH0 = NPT0 // CH
    NCH1 = NPT1 // CH
    OC = O // 16

    nc, ns = 2, 16
    mesh = plsc.VectorSubcoreMesh(core_axis_name="c", subcore_axis_name="s",
                                  num_cores=nc, num_subcores=ns)

    def body(tbl_hbm, idx_hbm, supp_hbm, out_hbm, idx_v, *rest):
        rows = rest[0:NBUF]
        sbs = rest[NBUF:2 * NBUF]
        obs = rest[2 * NBUF:3 * NBUF]
        sems = rest[3 * NBUF:4 * NBUF]
        semos = rest[4 * NBUF:5 * NBUF]

        cid = lax.axis_index("c")
        sid = lax.axis_index("s")
        base = lax.select(cid == 0, sid * NPT0, ns * NPT0 + sid * NPT1)
        nch = lax.select(cid == 0, NCH0, NCH1)
        trips = lax.select(cid == 0, NCH0 // NBUF, NCH1 // NBUF)

        @pl.when(cid == 0)
        def _():
            pltpu.sync_copy(idx_hbm.at[pl.ds(base * K2, NPT0 * K2)],
                            idx_v.at[pl.ds(0, NPT0 * K2)])

        @pl.when(cid != 0)
        def _():
            pltpu.sync_copy(idx_hbm.at[pl.ds(base * K2, NPT1 * K2)],
                            idx_v.at[pl.ds(0, NPT1 * K2)])

        NS_G = GC // 128

        def fetch(c, b, do):
            ds = [pltpu.make_async_copy(
                      tbl_hbm.at[idx_v.at[pl.ds(c * GC + i * 128, 128)]],
                      rows[b].at[pl.ds(i * 128, 128)], sems[b])
                  for i in range(NS_G)]
            ds.append(pltpu.make_async_copy(
                supp_hbm.at[pl.ds((base + c * CH) * K, CH * K)], sbs[b],
                sems[b]))
            for d in ds:
                d.start() if do == "start" else d.wait()

        for b in range(NBUF):
            fetch(b, b, "start")

        def outcopy(c, b):
            return pltpu.make_async_copy(
                obs[b], out_hbm.at[pl.ds(base + c * CH, CH)], semos[b])

        def compute_chunk(rv, sv, ov):
            for j in range(CH):
                rbase = j * K2

                def kbody(k, accs):
                    s = sv[j * K + k, :]
                    out = []
                    for oc in range(OC):
                        u = rv[rbase + k, pl.ds(oc * 16, 16)]
                        v = rv[rbase + K + k, pl.ds(oc * 16, 16)]
                        f = jnp.maximum(u + v, 0.0) * s
                        out.append(jnp.maximum(accs[oc], f))
                    return tuple(out)

                accs = lax.fori_loop(
                    0, K, kbody,
                    tuple(jnp.zeros((16,), jnp.float32) for _ in range(OC)))
                for oc in range(OC):
                    ov[j, pl.ds(oc * 16, 16)] = accs[oc]

        def outer(it, carry):
            for b in range(NBUF):
                c = it * NBUF + b
                fetch(c, b, "wait")

                @pl.when(c >= NBUF)
                def _():
                    outcopy(c - NBUF, b).wait()

                compute_chunk(rows[b], sbs[b], obs[b])
                outcopy(c, b).start()

                @pl.when(c + NBUF < nch)
                def _():
                    fetch(c + NBUF, b, "start")

            return carry

        lax.fori_loop(0, trips, outer, 0)
        for b in range(NBUF):
            outcopy(nch - NBUF + b, b).wait()

    return functools.partial(
        pl.kernel,
        out_type=jax.ShapeDtypeStruct((NPAD, O), jnp.float32),
        mesh=mesh,
        scratch_types=(
            [pltpu.VMEM((NPT0 * K2,), jnp.int32)]
            + [pltpu.VMEM((GC, O), jnp.float32) for _ in range(NBUF)]
            + [pltpu.VMEM((CH * K, 16), jnp.float32) for _ in range(NBUF)]
            + [pltpu.VMEM((CH, O), jnp.float32) for _ in range(NBUF)]
            + [pltpu.SemaphoreType.DMA for _ in range(2 * NBUF)]
        ),
    )(body)


def kernel(x, edge_index, pos, dis, W, b, att_W, att_b):
    del pos
    B, C, N, _ = x.shape
    K = edge_index.shape[-1]
    O = W.shape[0]
    NS = 16
    CH = 4
    NBUF = 4
    QUANT = 2 * NS * CH * NBUF
    NPAD = ((N + QUANT - 1) // QUANT) * QUANT
    BN = NPAD // 4
    NT = NPAD // NS
    NPT0 = ((NT * 4) // (5 * CH * NBUF)) * (CH * NBUF)
    NPT1 = NT - NPT0

    X = x[0, :, :, 0]
    xp = jnp.pad(X, ((0, 0), (0, NPAD - N)))
    disp = jnp.pad(dis[0], ((0, NPAD - N), (0, 0)))
    i1 = edge_index[1, 0].astype(jnp.int32)
    i0 = edge_index[0, 0].astype(jnp.int32)
    idx = jnp.concatenate([i1, i0 + NPAD], axis=1)
    idxp = jnp.pad(idx, ((0, NPAD - N), (0, 0))).reshape(-1)

    W1, W2 = W[:, :C], W[:, C:]
    wst = jnp.stack([(W1 - W2).T, W2.T])
    bst = jnp.stack([b[None, :], jnp.zeros((1, O), jnp.float32)])
    aw = att_W[0].reshape(K, 1).astype(jnp.float32)
    ab = att_b.reshape(1, 1).astype(jnp.float32)
    ee = jnp.repeat(jnp.eye(K, dtype=jnp.float32), 16, axis=1)

    tbl, supp = _tc_tables(xp, wst, bst, disp, aw, ab, ee,
                           C=C, O=O, K=K, NPAD=NPAD, BN=BN)
    out = _make_sc_combine(O=O, K=K, NPAD=NPAD, CH=CH, NBUF=NBUF,
                           NPT0=NPT0, NPT1=NPT1)(
        tbl, idxp, supp.reshape(NPAD * K, 16))
    return out[:N].T[None, :, :, None]
